# baseline (speedup 1.0000x reference)
import functools

import jax
import jax.numpy as jnp
from jax import lax
from jax.experimental import pallas as pl
from jax.experimental.pallas import tpu as pltpu
from jax.experimental.pallas import tpu_sc as plsc

_B, _H, _W, _C = 16, 512, 512, 8
_HW = _H * _W
_NPAD = 512
_NW = 32
_RPW = _NPAD // 2
_EPW = _RPW * _C
_NCH = _EPW // 128


def _make_gather():
    mesh = plsc.VectorSubcoreMesh(core_axis_name="c", subcore_axis_name="s")

    @functools.partial(
        pl.kernel,
        mesh=mesh,
        out_type=jax.ShapeDtypeStruct((_NW * _EPW,), jnp.float32),
        scratch_types=[
            pltpu.VMEM((_RPW,), jnp.int32),
            pltpu.VMEM((_NCH, 128), jnp.int32),
            pltpu.VMEM((_EPW,), jnp.float32),
            pltpu.SemaphoreType.DMA,
        ],
        compiler_params=pltpu.CompilerParams(needs_layout_passes=False),
    )
    def gather_kernel(table_hbm, idx_hbm, out_hbm, idx_v, eidx_v, rows_v, sem):
        wid = lax.axis_index("s") * 2 + lax.axis_index("c")
        batch = wid // 2
        pltpu.sync_copy(idx_hbm.at[wid], idx_v)
        lane = lax.iota(jnp.int32, 16)
        rsel = lax.shift_right_logical(lane, 3)
        chan = lax.bitwise_and(lane, 7)
        base = chan + batch * (_HW * _C)
        for k in range(_EPW // 16):
            rvec = plsc.load_gather(idx_v, [rsel + (2 * k)])
            eidx_v[k // 8, pl.ds((k % 8) * 16, 16)] = (
                lax.shift_left(rvec, 3) + base)
        cps = [
            pltpu.async_copy(
                table_hbm.at[eidx_v.at[j]],
                rows_v.at[pl.ds(j * 128, 128)], sem)
            for j in range(_NCH)
        ]
        for cp in cps:
            cp.wait()
        pltpu.sync_copy(rows_v, out_hbm.at[pl.ds(wid * _EPW, _EPW)])

    return gather_kernel


_gather = _make_gather()


def kernel(feat, index):
    B, H, W, C = feat.shape
    N = index.shape[1]
    table = feat.reshape(B * H * W * C)
    idx = jnp.pad(index, ((0, 0), (0, _NPAD - N))).reshape(_NW, _RPW)
    out = _gather(table, idx)
    return out.reshape(B, _NPAD, C)[:, :N, :]
